# Initial kernel scaffold; baseline (speedup 1.0000x reference)
import jax
import jax.numpy as jnp
from jax.experimental import pallas as pl


def kernel(inputs, embedding_table, pos_emb_cache):
    raise NotImplementedError("write your pallas kernel here")


# speedup vs baseline: 4.7967x; 4.7967x over previous
import functools

import jax
import jax.numpy as jnp
from jax import lax
from jax.experimental import pallas as pl
from jax.experimental.pallas import tpu as pltpu
from jax.experimental.pallas import tpu_sc as plsc

_EMB_D = 128
_LANES = 16
_GROUPS = _EMB_D // _LANES


@functools.lru_cache(maxsize=None)
def _make_gather(num_rows: int, seq: int):
    info = plsc.get_sparse_core_info()
    nc, ns = info.num_cores, info.num_subcores
    nw = nc * ns
    chunk = 2 * seq
    n_chunks_total = num_rows // chunk
    n_chunks = n_chunks_total // nw
    assert num_rows % (chunk * nw) == 0

    mesh = plsc.VectorSubcoreMesh(core_axis_name="c", subcore_axis_name="s")

    @functools.partial(
        pl.kernel,
        out_type=jax.ShapeDtypeStruct((n_chunks_total, chunk, _EMB_D),
                                      jnp.float32),
        mesh=mesh,
        scratch_types=[
            pltpu.VMEM((n_chunks, chunk), jnp.int32),
            pltpu.VMEM((chunk, _EMB_D), jnp.float32),
            pltpu.VMEM((chunk, _EMB_D), jnp.float32),
            pltpu.SemaphoreType.DMA,
        ],
    )
    def gather_kernel(table_hbm, idx_hbm, pos_hbm, out_hbm,
                      idx_v, pos_v, rows_v, sem):
        wid = lax.axis_index("s") * nc + lax.axis_index("c")
        pltpu.sync_copy(idx_hbm.at[wid], idx_v)
        pltpu.sync_copy(pos_hbm, pos_v)
        for j in range(n_chunks):
            pltpu.async_copy(table_hbm.at[idx_v.at[j]], rows_v, sem).wait()

            @pl.loop(0, chunk)
            def _(r):
                for g in range(_GROUPS):
                    sl = pl.ds(g * _LANES, _LANES)
                    rows_v[r, sl] = rows_v[r, sl] + pos_v[r, sl]

            pltpu.sync_copy(rows_v, out_hbm.at[wid * n_chunks + j])

    def run(table, idx, posblk):
        idx3 = idx.reshape(nw, n_chunks, chunk)
        out = gather_kernel(table, idx3, posblk)
        return out.reshape(num_rows, _EMB_D)

    return run


def kernel(inputs, embedding_table, pos_emb_cache):
    bs, seq = inputs.shape
    vocab, d = embedding_table.shape
    gather = _make_gather(bs * seq, seq)
    posblk = jnp.tile(pos_emb_cache[:seq], (2, 1))
    x = gather(embedding_table, inputs.reshape(-1), posblk)
    x = x.reshape(bs, seq, d)
    pos_emb = jnp.broadcast_to(pos_emb_cache[None, :seq, :], (bs, seq, d))
    mask = (inputs > 0).astype(jnp.int32)
    attn_pattern_mask = jnp.ones((bs, 4, seq, seq), jnp.float32)
    modality_index = jnp.array(0, dtype=jnp.int32)
    return (x, pos_emb, modality_index, mask, attn_pattern_mask)


# speedup vs baseline: 7.5983x; 1.5841x over previous
import functools

import jax
import jax.numpy as jnp
from jax import lax
from jax.experimental import pallas as pl
from jax.experimental.pallas import tpu as pltpu
from jax.experimental.pallas import tpu_sc as plsc

_EMB_D = 128
_LANES = 16
_GROUPS = _EMB_D // _LANES
_NBUF = 4
_SPC = 2


@functools.lru_cache(maxsize=None)
def _make_gather(bs: int, seq: int):
    info = plsc.get_sparse_core_info()
    nc, ns = info.num_cores, info.num_subcores
    nw = nc * ns
    rows_per_w = bs // nw
    n_chunks = rows_per_w // _SPC
    assert bs % (nw * _SPC) == 0
    seq_pad = (seq + 7) // 8 * 8

    mesh = plsc.VectorSubcoreMesh(core_axis_name="c", subcore_axis_name="s")

    @functools.partial(
        pl.kernel,
        out_type=jax.ShapeDtypeStruct((bs, seq, _EMB_D), jnp.float32),
        mesh=mesh,
        scratch_types=[
            pltpu.VMEM((rows_per_w, seq), jnp.int32),
            pltpu.VMEM((seq_pad, _EMB_D), jnp.float32),
            pltpu.VMEM((_NBUF, _SPC, seq, _EMB_D), jnp.float32),
            pltpu.SemaphoreType.DMA((_NBUF,)),
            pltpu.SemaphoreType.DMA((_NBUF,)),
        ],
    )
    def gather_kernel(table_hbm, idx_hbm, pos_hbm, out_hbm,
                      idx_v, pos_v, buf, gsem, wsem):
        wid = lax.axis_index("s") * nc + lax.axis_index("c")
        base = wid * rows_per_w
        pltpu.sync_copy(idx_hbm.at[wid], idx_v)
        pltpu.sync_copy(pos_hbm.at[pl.ds(0, seq_pad)], pos_v)

        def issue_gather(j):
            p = j % _NBUF
            return tuple(
                pltpu.async_copy(table_hbm.at[idx_v.at[_SPC * j + s]],
                                 buf.at[p, s], gsem.at[p])
                for s in range(_SPC)
            )

        gdesc = [None] * n_chunks
        wdesc = [None] * n_chunks
        gdesc[0] = issue_gather(0)
        if n_chunks > 1:
            gdesc[1] = issue_gather(1)
        for j in range(n_chunks):
            p = j % _NBUF
            for d in gdesc[j]:
                d.wait()
            for s in range(_SPC):
                @pl.loop(0, seq)
                def _(r, s=s):
                    for g in range(_GROUPS):
                        sl = pl.ds(g * _LANES, _LANES)
                        plsc.addupdate(buf.at[p, s, r, sl], pos_v[r, sl])
            if j + 2 < n_chunks:
                if j >= 2:
                    wdesc[j - 2].wait()
                gdesc[j + 2] = issue_gather(j + 2)
            wdesc[j] = pltpu.async_copy(
                buf.at[p], out_hbm.at[pl.ds(base + _SPC * j, _SPC)],
                wsem.at[p])
        for j in range(max(n_chunks - 2, 0), n_chunks):
            wdesc[j].wait()

    def run(table, idx, pos):
        idx3 = idx.reshape(nw, rows_per_w, seq)
        return gather_kernel(table, idx3, pos)

    return run


def kernel(inputs, embedding_table, pos_emb_cache):
    bs, seq = inputs.shape
    vocab, d = embedding_table.shape
    gather = _make_gather(bs, seq)
    x = gather(embedding_table, inputs, pos_emb_cache)
    pos_emb = jnp.broadcast_to(pos_emb_cache[None, :seq, :], (bs, seq, d))
    mask = (inputs > 0).astype(jnp.int32)
    attn_pattern_mask = jnp.ones((bs, 4, seq, seq), jnp.float32)
    modality_index = jnp.array(0, dtype=jnp.int32)
    return (x, pos_emb, modality_index, mask, attn_pattern_mask)


# speedup vs baseline: 9.7892x; 1.2883x over previous
import functools

import jax
import jax.numpy as jnp
from jax import lax
from jax.experimental import pallas as pl
from jax.experimental.pallas import tpu as pltpu
from jax.experimental.pallas import tpu_sc as plsc

_EMB_D = 128
_LANES = 16
_GROUPS = _EMB_D // _LANES
_NBUF = 4
_CHUNK = 64


@functools.lru_cache(maxsize=None)
def _make_gather(bs: int, seq: int):
    info = plsc.get_sparse_core_info()
    nc, ns = info.num_cores, info.num_subcores
    nw = nc * ns
    total = bs * seq
    n_chunks_total = total // _CHUNK
    n_chunks = n_chunks_total // nw
    chunks_per_s = bs // _CHUNK
    assert total % (_CHUNK * nw) == 0 and bs % _CHUNK == 0
    seq_pad = (seq + 7) // 8 * 8

    mesh = plsc.VectorSubcoreMesh(core_axis_name="c", subcore_axis_name="s")

    @functools.partial(
        pl.kernel,
        out_type=jax.ShapeDtypeStruct((total, _EMB_D), jnp.float32),
        mesh=mesh,
        scratch_types=[
            pltpu.VMEM((n_chunks, _CHUNK), jnp.int32),
            pltpu.VMEM((seq_pad, _EMB_D), jnp.float32),
            pltpu.VMEM((_NBUF, _CHUNK, _EMB_D), jnp.float32),
            pltpu.SemaphoreType.DMA((_NBUF,)),
            pltpu.SemaphoreType.DMA((_NBUF,)),
        ],
    )
    def gather_kernel(table_hbm, idx_hbm, pos_hbm, out_hbm,
                      idx_v, pos_v, buf, gsem, wsem):
        wid = lax.axis_index("s") * nc + lax.axis_index("c")
        c0 = wid * n_chunks
        pltpu.sync_copy(idx_hbm.at[wid], idx_v)
        pltpu.sync_copy(pos_hbm.at[pl.ds(0, seq_pad)], pos_v)

        def issue_gather(j):
            p = j % _NBUF
            return pltpu.async_copy(table_hbm.at[idx_v.at[j]], buf.at[p],
                                    gsem.at[p])

        gdesc = [None] * n_chunks
        wdesc = [None] * n_chunks
        gdesc[0] = issue_gather(0)
        gdesc[1] = issue_gather(1)
        for j in range(n_chunks):
            p = j % _NBUF
            gdesc[j].wait()
            s = (c0 + j) // chunks_per_s
            prow = [pos_v[s, pl.ds(g * _LANES, _LANES)] for g in range(_GROUPS)]

            @pl.loop(0, _CHUNK)
            def _(r):
                for g in range(_GROUPS):
                    plsc.addupdate(buf.at[p, r, pl.ds(g * _LANES, _LANES)],
                                   prow[g])

            if j + 2 < n_chunks:
                if j >= 2:
                    wdesc[j - 2].wait()
                gdesc[j + 2] = issue_gather(j + 2)
            wdesc[j] = pltpu.async_copy(
                buf.at[p],
                out_hbm.at[pl.ds((c0 + j) * _CHUNK, _CHUNK)],
                wsem.at[p])
        for j in range(n_chunks - 2, n_chunks):
            wdesc[j].wait()

    def run(table, ids, pos):
        ids_smajor = ids.T.reshape(nw, n_chunks, _CHUNK)
        out = gather_kernel(table, ids_smajor, pos)
        return out.reshape(seq, bs, _EMB_D).transpose(1, 0, 2)

    return run


def kernel(inputs, embedding_table, pos_emb_cache):
    bs, seq = inputs.shape
    vocab, d = embedding_table.shape
    gather = _make_gather(bs, seq)
    x = gather(embedding_table, inputs, pos_emb_cache)
    pos_emb = jnp.broadcast_to(pos_emb_cache[None, :seq, :], (bs, seq, d))
    mask = (inputs > 0).astype(jnp.int32)
    attn_pattern_mask = jnp.ones((bs, 4, seq, seq), jnp.float32)
    modality_index = jnp.array(0, dtype=jnp.int32)
    return (x, pos_emb, modality_index, mask, attn_pattern_mask)


# speedup vs baseline: 10.4030x; 1.0627x over previous
import functools

import jax
import jax.numpy as jnp
from jax import lax
from jax.experimental import pallas as pl
from jax.experimental.pallas import tpu as pltpu
from jax.experimental.pallas import tpu_sc as plsc

_EMB_D = 128
_LANES = 16
_GROUPS = _EMB_D // _LANES
_NBUF = 6
_CHUNK = 64


@functools.lru_cache(maxsize=None)
def _make_gather(bs: int, seq: int):
    info = plsc.get_sparse_core_info()
    nc, ns = info.num_cores, info.num_subcores
    nw = nc * ns
    total = bs * seq
    n_chunks_total = total // _CHUNK
    n_chunks = n_chunks_total // nw
    chunks_per_s = bs // _CHUNK
    assert total % (_CHUNK * nw) == 0 and bs % _CHUNK == 0
    seq_pad = (seq + 7) // 8 * 8

    mesh = plsc.VectorSubcoreMesh(core_axis_name="c", subcore_axis_name="s")

    @functools.partial(
        pl.kernel,
        out_type=jax.ShapeDtypeStruct((total, _EMB_D), jnp.float32),
        mesh=mesh,
        cost_estimate=pl.CostEstimate(
            flops=2 * total * _EMB_D,
            bytes_accessed=2 * total * _EMB_D * 4,
            transcendentals=0,
        ),
        scratch_types=[
            pltpu.VMEM((n_chunks, _CHUNK), jnp.int32),
            pltpu.VMEM((seq_pad, _EMB_D), jnp.float32),
            pltpu.VMEM((_NBUF, _CHUNK, _EMB_D), jnp.float32),
            pltpu.SemaphoreType.DMA((_NBUF,)),
            pltpu.SemaphoreType.DMA((_NBUF,)),
        ],
    )
    def gather_kernel(table_hbm, idx_hbm, pos_hbm, out_hbm,
                      idx_v, pos_v, buf, gsem, wsem):
        wid = lax.axis_index("s") * nc + lax.axis_index("c")
        c0 = wid * n_chunks
        pltpu.sync_copy(idx_hbm.at[wid], idx_v)
        pltpu.sync_copy(pos_hbm.at[pl.ds(0, seq_pad)], pos_v)

        def issue_gather(j):
            p = j % _NBUF
            return pltpu.async_copy(table_hbm.at[idx_v.at[j]], buf.at[p],
                                    gsem.at[p])

        gdesc = [None] * n_chunks
        wdesc = [None] * n_chunks
        gdesc[0] = issue_gather(0)
        gdesc[1] = issue_gather(1)
        gdesc[2] = issue_gather(2)
        for j in range(n_chunks):
            p = j % _NBUF
            gdesc[j].wait()
            s = (c0 + j) // chunks_per_s
            prow = [pos_v[s, pl.ds(g * _LANES, _LANES)] for g in range(_GROUPS)]

            @pl.loop(0, _CHUNK)
            def _(r):
                for g in range(_GROUPS):
                    plsc.addupdate(buf.at[p, r, pl.ds(g * _LANES, _LANES)],
                                   prow[g])

            if j + 3 < n_chunks:
                if j >= 3:
                    wdesc[j - 3].wait()
                gdesc[j + 3] = issue_gather(j + 3)
            wdesc[j] = pltpu.async_copy(
                buf.at[p],
                out_hbm.at[pl.ds((c0 + j) * _CHUNK, _CHUNK)],
                wsem.at[p])
        for j in range(n_chunks - 3, n_chunks):
            wdesc[j].wait()

    def run(table, ids, pos):
        ids_smajor = ids.T.reshape(nw, n_chunks, _CHUNK)
        out = gather_kernel(table, ids_smajor, pos)
        return out.reshape(seq, bs, _EMB_D).transpose(1, 0, 2)

    return run


def kernel(inputs, embedding_table, pos_emb_cache):
    bs, seq = inputs.shape
    vocab, d = embedding_table.shape
    gather = _make_gather(bs, seq)
    x = gather(embedding_table, inputs, pos_emb_cache)
    pos_emb = jnp.broadcast_to(pos_emb_cache[None, :seq, :], (bs, seq, d))
    mask = (inputs > 0).astype(jnp.int32)
    attn_pattern_mask = jnp.ones((bs, 4, seq, seq), jnp.float32)
    modality_index = jnp.array(0, dtype=jnp.int32)
    return (x, pos_emb, modality_index, mask, attn_pattern_mask)


# speedup vs baseline: 11.4812x; 1.1036x over previous
import functools

import jax
import jax.numpy as jnp
from jax import lax
from jax.experimental import pallas as pl
from jax.experimental.pallas import tpu as pltpu
from jax.experimental.pallas import tpu_sc as plsc

_EMB_D = 128
_LANES = 16
_GROUPS = _EMB_D // _LANES
_NBUF = 6
_CHUNK = 64


@functools.lru_cache(maxsize=None)
def _make_gather(bs: int, seq: int):
    info = plsc.get_sparse_core_info()
    nc, ns = info.num_cores, info.num_subcores
    nw = nc * ns
    total = bs * seq
    n_chunks_total = total // _CHUNK
    n_chunks = n_chunks_total // nw
    chunks_per_s = bs // _CHUNK
    assert total % (_CHUNK * nw) == 0 and bs % _CHUNK == 0
    seq_pad = (seq + 7) // 8 * 8

    mesh = plsc.VectorSubcoreMesh(core_axis_name="c", subcore_axis_name="s")

    @functools.partial(
        pl.kernel,
        out_type=jax.ShapeDtypeStruct((total, _EMB_D), jnp.float32),
        mesh=mesh,
        cost_estimate=pl.CostEstimate(
            flops=2 * total * _EMB_D,
            bytes_accessed=2 * total * _EMB_D * 4,
            transcendentals=0,
        ),
        scratch_types=[
            pltpu.VMEM((n_chunks, _CHUNK), jnp.int32),
            pltpu.VMEM((seq_pad, _EMB_D), jnp.float32),
            pltpu.VMEM((_NBUF, _CHUNK, _EMB_D), jnp.float32),
            pltpu.SemaphoreType.DMA((_NBUF,)),
            pltpu.SemaphoreType.DMA((_NBUF,)),
        ],
    )
    def gather_kernel(table_hbm, idx_hbm, pos_hbm, out_hbm,
                      idx_v, pos_v, buf, gsem, wsem):
        wid = lax.axis_index("s") * nc + lax.axis_index("c")
        c0 = wid * n_chunks
        pltpu.sync_copy(idx_hbm.at[wid], idx_v)
        pltpu.sync_copy(pos_hbm.at[pl.ds(0, seq_pad)], pos_v)

        def issue_gather(j):
            p = j % _NBUF
            return pltpu.async_copy(table_hbm.at[idx_v.at[j]], buf.at[p],
                                    gsem.at[p])

        gdesc = [None] * n_chunks
        wdesc = [None] * n_chunks
        gdesc[0] = issue_gather(0)
        gdesc[1] = issue_gather(1)
        gdesc[2] = issue_gather(2)
        for j in range(n_chunks):
            p = j % _NBUF
            gdesc[j].wait()
            s = (c0 + j) // chunks_per_s
            prow = [pos_v[s, pl.ds(g * _LANES, _LANES)] for g in range(_GROUPS)]

            @pl.loop(0, _CHUNK)
            def _(r):
                for g in range(_GROUPS):
                    plsc.addupdate(buf.at[p, r, pl.ds(g * _LANES, _LANES)],
                                   prow[g])

            if j + 3 < n_chunks:
                if j >= 3:
                    wdesc[j - 3].wait()
                gdesc[j + 3] = issue_gather(j + 3)
            wdesc[j] = pltpu.async_copy(
                buf.at[p],
                out_hbm.at[pl.ds((c0 + j) * _CHUNK, _CHUNK)],
                wsem.at[p])
        for j in range(n_chunks - 3, n_chunks):
            wdesc[j].wait()

    def run(table, ids, pos):
        ids_smajor = ids.T.reshape(nw, n_chunks, _CHUNK)
        out = gather_kernel(table, ids_smajor, pos)
        return out.reshape(seq, bs, _EMB_D).transpose(1, 0, 2)

    return run


def kernel(inputs, embedding_table, pos_emb_cache):
    bs, seq = inputs.shape
    vocab, d = embedding_table.shape
    gather = _make_gather(bs, seq)
    rv = (inputs[0, :] >= 0).astype(jnp.float32)
    zv = (inputs[:, 0] < 0).astype(jnp.float32)
    pos_emb = (jnp.broadcast_to(pos_emb_cache[None, :seq, :], (bs, seq, d))
               + jnp.broadcast_to(zv[:, None, None], (bs, seq, d)))
    mask = (inputs > 0).astype(jnp.int32)
    attn_pattern_mask = jnp.maximum(
        jnp.broadcast_to(rv[None, None, :, None], (bs, 4, seq, seq)),
        jnp.broadcast_to(rv[None, None, None, :], (bs, 4, seq, seq)))
    x = gather(embedding_table, inputs, pos_emb_cache)
    modality_index = jnp.array(0, dtype=jnp.int32)
    return (x, pos_emb, modality_index, mask, attn_pattern_mask)
